# baseline (speedup 1.0000x reference)
import functools
import math

import jax
import jax.numpy as jnp
import numpy as np
from jax import lax
from jax.experimental import pallas as pl
from jax.experimental.pallas import tpu as pltpu
from jax.experimental.pallas import tpu_sc as plsc

VOCAB = 1000000
EMB_DIM = 64
CONTEXT = 200
BATCH = 1024

_NUM_CORES = 2
_NUM_SUBCORES = 16
_NW = _NUM_CORES * _NUM_SUBCORES
_BF = BATCH * CONTEXT
_PER_W = _BF // _NW
_CHUNK = 128
_NCH = _PER_W // _CHUNK
_VPR = EMB_DIM // 16


def _pos_encoding() -> np.ndarray:
    pos = np.arange(CONTEXT, dtype=np.float32)[:, None]
    i = np.arange(EMB_DIM)[None, :]
    angle_rates = np.power(10000.0, (2 * (i // 2)).astype(np.float32) / float(EMB_DIM))
    angles = pos / angle_rates
    pe = np.where(i % 2 == 0, np.sin(angles), np.cos(angles))
    return pe.astype(np.float32)


_PE = _pos_encoding()


def _emb_body(table_hbm, x_hbm, pe_hbm, out_hbm, idx_v, rows_v, pe_v, sem):
    wid = lax.axis_index("s") * _NUM_CORES + lax.axis_index("c")
    base_w = wid * _PER_W
    pltpu.sync_copy(pe_hbm, pe_v)

    def chunk_body(c, carry):
        base = base_w + c * _CHUNK
        pltpu.sync_copy(x_hbm.at[pl.ds(base, _CHUNK)], idx_v)
        pltpu.async_copy(table_hbm.at[idx_v], rows_v, sem).wait()
        phase = lax.rem(c * _CHUNK, CONTEXT)

        def row_body(r, _):
            l = phase + r
            l = jnp.where(l >= CONTEXT, l - CONTEXT, l)
            for j in range(_VPR):
                sl = pl.ds(j * 16, 16)
                rows_v[r, sl] = rows_v[r, sl] + pe_v[l, sl]
            return 0

        lax.fori_loop(0, _CHUNK, row_body, 0)
        pltpu.sync_copy(rows_v, out_hbm.at[pl.ds(base, _CHUNK)])
        return carry

    lax.fori_loop(0, _NCH, chunk_body, 0)


@jax.jit
def kernel(x, table):
    xf = x.reshape(-1).astype(jnp.int32)
    pe = jnp.asarray(_PE)
    mesh = plsc.VectorSubcoreMesh(core_axis_name="c", subcore_axis_name="s")
    run = functools.partial(
        pl.kernel,
        mesh=mesh,
        compiler_params=pltpu.CompilerParams(use_tc_tiling_on_sc=False),
        out_type=jax.ShapeDtypeStruct((_BF, EMB_DIM), jnp.float32),
        scratch_types=[
            pltpu.VMEM((_CHUNK,), jnp.int32),
            pltpu.VMEM((_CHUNK, EMB_DIM), jnp.float32),
            pltpu.VMEM((CONTEXT, EMB_DIM), jnp.float32),
            pltpu.SemaphoreType.DMA,
        ],
    )(_emb_body)
    out = run(table, xf, pe)
    return out.reshape(BATCH, CONTEXT, EMB_DIM)


# speedup vs baseline: 1.1809x; 1.1809x over previous
import functools
import math

import jax
import jax.numpy as jnp
import numpy as np
from jax import lax
from jax.experimental import pallas as pl
from jax.experimental.pallas import tpu as pltpu
from jax.experimental.pallas import tpu_sc as plsc

VOCAB = 1000000
EMB_DIM = 64
CONTEXT = 200
BATCH = 1024

_NUM_CORES = 2
_NUM_SUBCORES = 16
_NW = _NUM_CORES * _NUM_SUBCORES
_BF = BATCH * CONTEXT
_PER_W = _BF // _NW
_CHUNK = CONTEXT
_NCH = _PER_W // _CHUNK
_VPR = EMB_DIM // 16
_UNROLL = 4


def _pos_encoding() -> np.ndarray:
    pos = np.arange(CONTEXT, dtype=np.float32)[:, None]
    i = np.arange(EMB_DIM)[None, :]
    angle_rates = np.power(10000.0, (2 * (i // 2)).astype(np.float32) / float(EMB_DIM))
    angles = pos / angle_rates
    pe = np.where(i % 2 == 0, np.sin(angles), np.cos(angles))
    return pe.astype(np.float32)


_PE = _pos_encoding()


def _emb_body(table_hbm, x_hbm, pe_hbm, out_hbm,
              idx0, idx1, rows0, rows1, pe_v, g0, g1, o0, o1):
    wid = lax.axis_index("s") * _NUM_CORES + lax.axis_index("c")
    base_w = wid * _PER_W
    idx_b = (idx0, idx1)
    rows_b = (rows0, rows1)
    g_sem = (g0, g1)
    o_sem = (o0, o1)

    pltpu.sync_copy(pe_hbm, pe_v)

    def start_gather(b, c):
        base = base_w + c * _CHUNK
        pltpu.sync_copy(x_hbm.at[pl.ds(base, _CHUNK)], idx_b[b])
        pltpu.make_async_copy(table_hbm.at[idx_b[b]], rows_b[b], g_sem[b]).start()

    def wait_gather(b):
        pltpu.make_async_copy(table_hbm.at[idx_b[b]], rows_b[b], g_sem[b]).wait()

    def add_pe(b):
        rows_v = rows_b[b]

        def row_body(r4, _):
            for k in range(_UNROLL):
                r = r4 * _UNROLL + k
                for j in range(_VPR):
                    sl = pl.ds(j * 16, 16)
                    rows_v[r, sl] = rows_v[r, sl] + pe_v[r, sl]
            return 0

        lax.fori_loop(0, _CHUNK // _UNROLL, row_body, 0)

    def start_store(b, c):
        base = base_w + c * _CHUNK
        pltpu.make_async_copy(rows_b[b], out_hbm.at[pl.ds(base, _CHUNK)],
                              o_sem[b]).start()

    def wait_store(b, c):
        base = base_w + c * _CHUNK
        pltpu.make_async_copy(rows_b[b], out_hbm.at[pl.ds(base, _CHUNK)],
                              o_sem[b]).wait()

    start_gather(0, 0)
    start_gather(1, 1)

    def pair_body(t, _):
        for b in range(2):
            c = 2 * t + b
            wait_gather(b)
            add_pe(b)
            start_store(b, c)

            @pl.when(c + 2 < _NCH)
            def _prepare():
                wait_store(b, c)
                start_gather(b, c + 2)
            del _prepare
        return 0

    lax.fori_loop(0, _NCH // 2, pair_body, 0)
    wait_store(0, _NCH - 2)
    wait_store(1, _NCH - 1)


@jax.jit
def kernel(x, table):
    xf = x.reshape(-1).astype(jnp.int32)
    pe = jnp.asarray(_PE)
    mesh = plsc.VectorSubcoreMesh(core_axis_name="c", subcore_axis_name="s")
    run = functools.partial(
        pl.kernel,
        mesh=mesh,
        compiler_params=pltpu.CompilerParams(use_tc_tiling_on_sc=False),
        out_type=jax.ShapeDtypeStruct((_BF, EMB_DIM), jnp.float32),
        scratch_types=[
            pltpu.VMEM((_CHUNK,), jnp.int32),
            pltpu.VMEM((_CHUNK,), jnp.int32),
            pltpu.VMEM((_CHUNK, EMB_DIM), jnp.float32),
            pltpu.VMEM((_CHUNK, EMB_DIM), jnp.float32),
            pltpu.VMEM((CONTEXT, EMB_DIM), jnp.float32),
            pltpu.SemaphoreType.DMA,
            pltpu.SemaphoreType.DMA,
            pltpu.SemaphoreType.DMA,
            pltpu.SemaphoreType.DMA,
        ],
    )(_emb_body)
    out = run(table, xf, pe)
    return out.reshape(BATCH, CONTEXT, EMB_DIM)


# speedup vs baseline: 1.1830x; 1.0018x over previous
import functools
import math

import jax
import jax.numpy as jnp
import numpy as np
from jax import lax
from jax.experimental import pallas as pl
from jax.experimental.pallas import tpu as pltpu
from jax.experimental.pallas import tpu_sc as plsc

VOCAB = 1000000
EMB_DIM = 64
CONTEXT = 200
BATCH = 1024

_NUM_CORES = 2
_NUM_SUBCORES = 16
_NW = _NUM_CORES * _NUM_SUBCORES
_SEQ_PER_W = BATCH // _NW
_VPR = EMB_DIM // 16
_UNROLL = 4


def _pos_encoding() -> np.ndarray:
    pos = np.arange(CONTEXT, dtype=np.float32)[:, None]
    i = np.arange(EMB_DIM)[None, :]
    angle_rates = np.power(10000.0, (2 * (i // 2)).astype(np.float32) / float(EMB_DIM))
    angles = pos / angle_rates
    pe = np.where(i % 2 == 0, np.sin(angles), np.cos(angles))
    return pe.astype(np.float32)


_PE = _pos_encoding()


def _emb_body(table_hbm, x_hbm, pe_hbm, out_hbm,
              idx0, idx1, rows0, rows1, pe_v, g0, g1, o0, o1):
    wid = lax.axis_index("s") * _NUM_CORES + lax.axis_index("c")
    seq_base = wid * _SEQ_PER_W
    idx_b = (idx0, idx1)
    rows_b = (rows0, rows1)
    g_sem = (g0, g1)
    o_sem = (o0, o1)

    pltpu.sync_copy(pe_hbm, pe_v)

    def start_gather(b, c):
        pltpu.sync_copy(x_hbm.at[seq_base + c], idx_b[b])
        pltpu.make_async_copy(table_hbm.at[idx_b[b]], rows_b[b], g_sem[b]).start()

    def wait_gather(b):
        pltpu.make_async_copy(table_hbm.at[idx_b[b]], rows_b[b], g_sem[b]).wait()

    def add_pe(b):
        rows_v = rows_b[b]

        def row_body(r4, _):
            for k in range(_UNROLL):
                r = r4 * _UNROLL + k
                for j in range(_VPR):
                    sl = pl.ds(j * 16, 16)
                    rows_v[r, sl] = rows_v[r, sl] + pe_v[r, sl]
            return 0

        lax.fori_loop(0, CONTEXT // _UNROLL, row_body, 0)

    def start_store(b, c):
        pltpu.make_async_copy(rows_b[b], out_hbm.at[seq_base + c], o_sem[b]).start()

    def wait_store(b, c):
        pltpu.make_async_copy(rows_b[b], out_hbm.at[seq_base + c], o_sem[b]).wait()

    start_gather(0, 0)
    start_gather(1, 1)

    def pair_body(t, _):
        for b in range(2):
            c = 2 * t + b
            wait_gather(b)
            add_pe(b)
            start_store(b, c)

            @pl.when(c + 2 < _SEQ_PER_W)
            def _prepare():
                wait_store(b, c)
                start_gather(b, c + 2)
            del _prepare
        return 0

    lax.fori_loop(0, _SEQ_PER_W // 2, pair_body, 0)
    wait_store(0, _SEQ_PER_W - 2)
    wait_store(1, _SEQ_PER_W - 1)


@jax.jit
def kernel(x, table):
    pe = jnp.asarray(_PE)
    mesh = plsc.VectorSubcoreMesh(core_axis_name="c", subcore_axis_name="s")
    run = functools.partial(
        pl.kernel,
        mesh=mesh,
        compiler_params=pltpu.CompilerParams(use_tc_tiling_on_sc=False),
        out_type=jax.ShapeDtypeStruct((BATCH, CONTEXT, EMB_DIM), jnp.float32),
        scratch_types=[
            pltpu.VMEM((CONTEXT,), jnp.int32),
            pltpu.VMEM((CONTEXT,), jnp.int32),
            pltpu.VMEM((CONTEXT, EMB_DIM), jnp.float32),
            pltpu.VMEM((CONTEXT, EMB_DIM), jnp.float32),
            pltpu.VMEM((CONTEXT, EMB_DIM), jnp.float32),
            pltpu.SemaphoreType.DMA,
            pltpu.SemaphoreType.DMA,
            pltpu.SemaphoreType.DMA,
            pltpu.SemaphoreType.DMA,
        ],
    )(_emb_body)
    return run(table, x, pe)
